# Initial kernel scaffold; baseline (speedup 1.0000x reference)
import jax
import jax.numpy as jnp
from jax.experimental import pallas as pl


def kernel(x, edge_index, W_in, b_in, W1, b1, W2, b2):
    raise NotImplementedError("write your pallas kernel here")


# speedup vs baseline: 14.5261x; 14.5261x over previous
import functools

import jax
import jax.numpy as jnp
from jax import lax
from jax.experimental import pallas as pl
from jax.experimental.pallas import tpu as pltpu
from jax.experimental.pallas import tpu_sc as plsc

N = 10000
E = 320000
D = 128

NC = 2
NS = 16
NW = NC * NS
EPW = E // NW
C = 128
FULL_ITERS = EPW // C
REM = EPW - FULL_ITERS * C
RPT = 624
TAIL = N - NS * RPT
DW = 128

_mesh = plsc.VectorSubcoreMesh(core_axis_name="c", subcore_axis_name="s")


@functools.partial(
    pl.kernel,
    out_type=jax.ShapeDtypeStruct((NC * N, DW), jnp.float32),
    mesh=_mesh,
    scratch_types=[
        pltpu.VMEM((C, DW), jnp.float32),
        pltpu.VMEM((REM, DW), jnp.float32),
        pltpu.VMEM((C,), jnp.int32),
        pltpu.VMEM((REM,), jnp.int32),
        pltpu.VMEM_SHARED((N, DW), jnp.float32),
    ],
)
def _deg_sc(dst_hbm, ones_hbm, zrow_hbm, out_hbm,
            ones_v, ones_r, idx_v, idx_r, acc):
    c = lax.axis_index("c")
    s = lax.axis_index("s")
    wid = c * NS + s
    base_e = wid * EPW

    pltpu.sync_copy(zrow_hbm.at[pl.ds(s * RPT, RPT)], acc.at[pl.ds(s * RPT, RPT)])
    @pl.when(s == 0)
    def _():
        pltpu.sync_copy(zrow_hbm.at[pl.ds(NS * RPT, TAIL)],
                        acc.at[pl.ds(NS * RPT, TAIL)])
    pltpu.sync_copy(ones_hbm, ones_v)
    pltpu.sync_copy(ones_hbm.at[pl.ds(0, REM)], ones_r)
    plsc.subcore_barrier()

    def step(i, carry):
        pltpu.sync_copy(dst_hbm.at[pl.ds(base_e + i * C, C)], idx_v)
        pltpu.sync_copy(ones_v, acc.at[idx_v], add=True)
        return carry

    lax.fori_loop(0, FULL_ITERS, step, 0)
    pltpu.sync_copy(dst_hbm.at[pl.ds(base_e + FULL_ITERS * C, REM)], idx_r)
    pltpu.sync_copy(ones_r, acc.at[idx_r], add=True)

    plsc.subcore_barrier()
    pltpu.sync_copy(acc.at[pl.ds(s * RPT, RPT)],
                    out_hbm.at[pl.ds(c * N + s * RPT, RPT)])
    @pl.when(s == 0)
    def _():
        pltpu.sync_copy(acc.at[pl.ds(NS * RPT, TAIL)],
                        out_hbm.at[pl.ds(c * N + NS * RPT, TAIL)])


@functools.partial(
    pl.kernel,
    out_type=jax.ShapeDtypeStruct((NC * N, D), jnp.float32),
    mesh=_mesh,
    scratch_types=[
        pltpu.VMEM((C,), jnp.int32),
        pltpu.VMEM((C,), jnp.int32),
        pltpu.VMEM((REM,), jnp.int32),
        pltpu.VMEM((REM,), jnp.int32),
        pltpu.VMEM((C, D), jnp.float32),
        pltpu.VMEM((REM, D), jnp.float32),
        pltpu.SemaphoreType.DMA,
        pltpu.VMEM_SHARED((N, D), jnp.float32),
    ],
)
def _edge_sc(src_hbm, dst_hbm, g_hbm, zeros_hbm, out_hbm,
             sidx_v, didx_v, sidx_r, didx_r, rows_v, rows_r, sem, acc):
    c = lax.axis_index("c")
    s = lax.axis_index("s")
    wid = c * NS + s
    base_e = wid * EPW

    pltpu.sync_copy(zeros_hbm.at[pl.ds(s * RPT, RPT)], acc.at[pl.ds(s * RPT, RPT)])
    @pl.when(s == 0)
    def _():
        pltpu.sync_copy(zeros_hbm.at[pl.ds(NS * RPT, TAIL)],
                        acc.at[pl.ds(NS * RPT, TAIL)])
    plsc.subcore_barrier()

    def step(i, carry):
        off = base_e + i * C
        pltpu.sync_copy(src_hbm.at[pl.ds(off, C)], sidx_v)
        pltpu.sync_copy(dst_hbm.at[pl.ds(off, C)], didx_v)
        pltpu.async_copy(g_hbm.at[sidx_v], rows_v, sem).wait()
        pltpu.sync_copy(rows_v, acc.at[didx_v], add=True)
        return carry

    lax.fori_loop(0, FULL_ITERS, step, 0)
    off = base_e + FULL_ITERS * C
    pltpu.sync_copy(src_hbm.at[pl.ds(off, REM)], sidx_r)
    pltpu.sync_copy(dst_hbm.at[pl.ds(off, REM)], didx_r)
    pltpu.async_copy(g_hbm.at[sidx_r], rows_r, sem).wait()
    pltpu.sync_copy(rows_r, acc.at[didx_r], add=True)

    plsc.subcore_barrier()
    pltpu.sync_copy(acc.at[pl.ds(s * RPT, RPT)],
                    out_hbm.at[pl.ds(c * N + s * RPT, RPT)])
    @pl.when(s == 0)
    def _():
        pltpu.sync_copy(acc.at[pl.ds(NS * RPT, TAIL)],
                        out_hbm.at[pl.ds(c * N + NS * RPT, TAIL)])


_R = 1000


def _proj_tc_body(x_ref, win_ref, bin_ref, w1_ref, c0_ref, c1_ref,
                  dis_ref, g0_ref):
    deg = 1.0 + c0_ref[...] + c1_ref[...]
    dis = lax.rsqrt(deg)
    h0 = jnp.dot(x_ref[...], win_ref[...],
                 preferred_element_type=jnp.float32) + bin_ref[...]
    g0_ref[...] = jnp.dot(h0, w1_ref[...],
                          preferred_element_type=jnp.float32) * dis
    dis_ref[...] = dis


def _proj_tc(x, w_in, b_in, w1, c0, c1):
    return pl.pallas_call(
        _proj_tc_body,
        grid=(N // _R,),
        in_specs=[
            pl.BlockSpec((_R, D), lambda i: (i, 0)),
            pl.BlockSpec((D, D), lambda i: (0, 0)),
            pl.BlockSpec((1, D), lambda i: (0, 0)),
            pl.BlockSpec((D, D), lambda i: (0, 0)),
            pl.BlockSpec((_R, 1), lambda i: (i, 0)),
            pl.BlockSpec((_R, 1), lambda i: (i, 0)),
        ],
        out_specs=[
            pl.BlockSpec((_R, 1), lambda i: (i, 0)),
            pl.BlockSpec((_R, D), lambda i: (i, 0)),
        ],
        out_shape=[
            jax.ShapeDtypeStruct((N, 1), jnp.float32),
            jax.ShapeDtypeStruct((N, D), jnp.float32),
        ],
    )(x, w_in, b_in, w1, c0, c1)


def _mid_tc_body(sa_ref, sb_ref, g_ref, dis_ref, b_ref, w_ref, gn_ref):
    dis = dis_ref[...]
    out = dis * (sa_ref[...] + sb_ref[...] + g_ref[...]) + b_ref[...]
    gn_ref[...] = jnp.dot(out, w_ref[...],
                          preferred_element_type=jnp.float32) * dis


def _mid_tc(sa, sb, g, dis, b, w):
    return pl.pallas_call(
        _mid_tc_body,
        grid=(N // _R,),
        in_specs=[
            pl.BlockSpec((_R, D), lambda i: (i, 0)),
            pl.BlockSpec((_R, D), lambda i: (i, 0)),
            pl.BlockSpec((_R, D), lambda i: (i, 0)),
            pl.BlockSpec((_R, 1), lambda i: (i, 0)),
            pl.BlockSpec((1, D), lambda i: (0, 0)),
            pl.BlockSpec((D, D), lambda i: (0, 0)),
        ],
        out_specs=pl.BlockSpec((_R, D), lambda i: (i, 0)),
        out_shape=jax.ShapeDtypeStruct((N, D), jnp.float32),
    )(sa, sb, g, dis, b, w)


def _final_tc_body(sa_ref, sb_ref, g_ref, dis_ref, b_ref, out_ref):
    out_ref[...] = dis_ref[...] * (sa_ref[...] + sb_ref[...] + g_ref[...]) \
        + b_ref[...]


def _final_tc(sa, sb, g, dis, b):
    return pl.pallas_call(
        _final_tc_body,
        grid=(N // _R,),
        in_specs=[
            pl.BlockSpec((_R, D), lambda i: (i, 0)),
            pl.BlockSpec((_R, D), lambda i: (i, 0)),
            pl.BlockSpec((_R, D), lambda i: (i, 0)),
            pl.BlockSpec((_R, 1), lambda i: (i, 0)),
            pl.BlockSpec((1, D), lambda i: (0, 0)),
        ],
        out_specs=pl.BlockSpec((_R, D), lambda i: (i, 0)),
        out_shape=jax.ShapeDtypeStruct((N, D), jnp.float32),
    )(sa, sb, g, dis, b)


def kernel(x, edge_index, W_in, b_in, W1, b1, W2, b2):
    src = edge_index[0]
    dst = edge_index[1]
    zeros = jnp.zeros((N, D), jnp.float32)
    zrow = jnp.zeros((N, DW), jnp.float32)
    ones = jnp.ones((C, DW), jnp.float32)

    cnt = _deg_sc(dst, ones, zrow)
    c0 = cnt[0:N, 0:1]
    c1 = cnt[N:2 * N, 0:1]

    dis, g0 = _proj_tc(x, W_in, b_in.reshape(1, D), W1, c0, c1)

    s0 = _edge_sc(src, dst, g0, zeros)
    g1 = _mid_tc(s0[0:N], s0[N:2 * N], g0, dis, b1.reshape(1, D), W2)

    s1 = _edge_sc(src, dst, g1, zeros)
    return _final_tc(s1[0:N], s1[N:2 * N], g1, dis, b2.reshape(1, D))
